# Initial kernel scaffold; baseline (speedup 1.0000x reference)
import jax
import jax.numpy as jnp
from jax.experimental import pallas as pl


def kernel(x, lut):
    raise NotImplementedError("write your pallas kernel here")


# speedup vs baseline: 2.9385x; 2.9385x over previous
import math

import jax
import jax.numpy as jnp
from jax import lax
from jax.experimental import pallas as pl
from jax.experimental.pallas import tpu as pltpu
from jax.experimental.pallas import tpu_sc as plsc

_VOCAB = 100000
_DIM = 128
_BATCH = 4096
_HIST = 50

_NC = 2
_NS = 16
_NW = _NC * _NS
_ROWS = _BATCH * _HIST
_CPW = _ROWS // _NW
_C = 128
_CHUNKS = _CPW // _C
_NBUF = 2
_LANES = 16
_SCALE = math.sqrt(_DIM)


def _emb_body(x_hbm, lut_hbm, out_hbm, idx_v, in_bufs, out_bufs, gsems, ssems):
    wid = lax.axis_index("s") * _NC + lax.axis_index("c")
    base = wid * _CPW

    pltpu.sync_copy(x_hbm.at[wid], idx_v)

    def gather(g, b):
        return pltpu.make_async_copy(lut_hbm.at[idx_v.at[g]], in_bufs[b], gsems[b])

    def store(g, b):
        dst = out_hbm.at[pl.ds(base + g * _C, _C)]
        return pltpu.make_async_copy(out_bufs[b], dst, ssems[b])

    for b in range(_NBUF):
        gather(b, b).start()

    def scale_chunk(b):
        src, dst = in_bufs[b], out_bufs[b]

        def row(r, carry):
            for j in range(_DIM // _LANES):
                sl = pl.ds(j * _LANES, _LANES)
                dst[r, sl] = src[r, sl] * _SCALE
            return carry

        lax.fori_loop(0, _C, row, 0)

    def outer(i, carry):
        for b in range(_NBUF):
            g = i * _NBUF + b
            gather(g, b).wait()

            @pl.when(g >= _NBUF)
            def _():
                store(g - _NBUF, b).wait()

            scale_chunk(b)

            @pl.when(g + _NBUF < _CHUNKS)
            def _():
                gather(g + _NBUF, b).start()

            store(g, b).start()
        return carry

    lax.fori_loop(0, _CHUNKS // _NBUF, outer, 0)

    for b in range(_NBUF):
        store(_CHUNKS - _NBUF + b, b).wait()


def _make_kernel():
    mesh = plsc.VectorSubcoreMesh(
        core_axis_name="c", subcore_axis_name="s",
        num_cores=_NC, num_subcores=_NS,
    )
    return pl.kernel(
        _emb_body,
        out_type=jax.ShapeDtypeStruct((_ROWS, _DIM), jnp.float32),
        mesh=mesh,
        scratch_types=[
            pltpu.VMEM((_CHUNKS, _C), jnp.int32),
            [pltpu.VMEM((_C, _DIM), jnp.float32) for _ in range(_NBUF)],
            [pltpu.VMEM((_C, _DIM), jnp.float32) for _ in range(_NBUF)],
            [pltpu.SemaphoreType.DMA for _ in range(_NBUF)],
            [pltpu.SemaphoreType.DMA for _ in range(_NBUF)],
        ],
    )


_emb_kernel = _make_kernel()


def kernel(x, lut):
    idx = x.astype(jnp.int32).reshape(_NW, _CHUNKS, _C)
    out = _emb_kernel(idx, lut)
    return out.reshape(_BATCH, _HIST, _DIM)


# speedup vs baseline: 5.2754x; 1.7953x over previous
import math

import jax
import jax.numpy as jnp
from jax import lax
from jax.experimental import pallas as pl
from jax.experimental.pallas import tpu as pltpu
from jax.experimental.pallas import tpu_sc as plsc

_VOCAB = 100000
_DIM = 128
_BATCH = 4096
_HIST = 50

_NC = 2
_NS = 16
_NW = _NC * _NS
_BPW = _BATCH // _NW
_NBUF = 4
_LANES = 16
_SCALE = math.sqrt(_DIM)


def _emb_body(x_hbm, lut_hbm, out_hbm, idx_v, in_bufs, out_bufs, gsems, ssems):
    wid = lax.axis_index("s") * _NC + lax.axis_index("c")
    base = wid * _BPW

    pltpu.sync_copy(x_hbm.at[pl.ds(base, _BPW)], idx_v)

    def gather(g, b):
        return pltpu.make_async_copy(lut_hbm.at[idx_v.at[g]], in_bufs[b], gsems[b])

    def store(g, b):
        return pltpu.make_async_copy(out_bufs[b], out_hbm.at[base + g], ssems[b])

    for b in range(_NBUF):
        gather(b, b).start()

    def scale_chunk(b):
        src, dst = in_bufs[b], out_bufs[b]

        def row(r, carry):
            for j in range(_DIM // _LANES):
                sl = pl.ds(j * _LANES, _LANES)
                dst[r, sl] = src[r, sl] * _SCALE
            return carry

        lax.fori_loop(0, _HIST, row, 0)

    def outer(i, carry):
        for b in range(_NBUF):
            g = i * _NBUF + b
            gather(g, b).wait()

            @pl.when(g >= _NBUF)
            def _():
                store(g - _NBUF, b).wait()

            scale_chunk(b)

            @pl.when(g + _NBUF < _BPW)
            def _():
                gather(g + _NBUF, b).start()

            store(g, b).start()
        return carry

    lax.fori_loop(0, _BPW // _NBUF, outer, 0)

    for b in range(_NBUF):
        store(_BPW - _NBUF + b, b).wait()


def _make_kernel():
    mesh = plsc.VectorSubcoreMesh(
        core_axis_name="c", subcore_axis_name="s",
        num_cores=_NC, num_subcores=_NS,
    )
    return pl.kernel(
        _emb_body,
        out_type=jax.ShapeDtypeStruct((_BATCH, _HIST, _DIM), jnp.float32),
        mesh=mesh,
        scratch_types=[
            pltpu.VMEM((_BPW, _HIST), jnp.int32),
            [pltpu.VMEM((_HIST, _DIM), jnp.float32) for _ in range(_NBUF)],
            [pltpu.VMEM((_HIST, _DIM), jnp.float32) for _ in range(_NBUF)],
            [pltpu.SemaphoreType.DMA for _ in range(_NBUF)],
            [pltpu.SemaphoreType.DMA for _ in range(_NBUF)],
        ],
    )


_emb_kernel = _make_kernel()


def kernel(x, lut):
    return _emb_kernel(x.astype(jnp.int32), lut)


# speedup vs baseline: 9.1580x; 1.7360x over previous
import math

import jax
import jax.numpy as jnp
from jax import lax
from jax.experimental import pallas as pl
from jax.experimental.pallas import tpu as pltpu
from jax.experimental.pallas import tpu_sc as plsc

_VOCAB = 100000
_DIM = 128
_BATCH = 4096
_HIST = 50

_NC = 2
_NS = 16
_NW = _NC * _NS
_ROWS = _BATCH * _HIST
_CPW = _ROWS // _NW
_C = 128
_CHUNKS = _CPW // _C
_BCHUNKS = _BATCH // _C
_NBUF = 2
_LANES = 16
_SCALE = math.sqrt(_DIM)


def _emb_body(x_hbm, lut_hbm, out_hbm, idx_v, in_bufs, out_bufs, gsems, ssems):
    wid = lax.axis_index("s") * _NC + lax.axis_index("c")

    pltpu.sync_copy(x_hbm.at[wid], idx_v)

    def gather(g, b):
        return pltpu.make_async_copy(lut_hbm.at[idx_v.at[g]], in_bufs[b], gsems[b])

    def store(g, b):
        gg = wid * _CHUNKS + g
        dst = out_hbm.at[gg // _BCHUNKS, pl.ds((gg % _BCHUNKS) * _C, _C)]
        return pltpu.make_async_copy(out_bufs[b], dst, ssems[b])

    for b in range(_NBUF):
        gather(b, b).start()

    def scale_chunk(b):
        src, dst = in_bufs[b], out_bufs[b]

        def row(r, carry):
            for j in range(_DIM // _LANES):
                sl = pl.ds(j * _LANES, _LANES)
                dst[r, sl] = src[r, sl] * _SCALE
            return carry

        lax.fori_loop(0, _C, row, 0)

    def outer(i, carry):
        for b in range(_NBUF):
            g = i * _NBUF + b
            gather(g, b).wait()

            @pl.when(g >= _NBUF)
            def _():
                store(g - _NBUF, b).wait()

            scale_chunk(b)

            @pl.when(g + _NBUF < _CHUNKS)
            def _():
                gather(g + _NBUF, b).start()

            store(g, b).start()
        return carry

    lax.fori_loop(0, _CHUNKS // _NBUF, outer, 0)

    for b in range(_NBUF):
        store(_CHUNKS - _NBUF + b, b).wait()


def _make_kernel():
    mesh = plsc.VectorSubcoreMesh(
        core_axis_name="c", subcore_axis_name="s",
        num_cores=_NC, num_subcores=_NS,
    )
    return pl.kernel(
        _emb_body,
        out_type=jax.ShapeDtypeStruct((_HIST, _BATCH, _DIM), jnp.float32),
        mesh=mesh,
        scratch_types=[
            pltpu.VMEM((_CHUNKS, _C), jnp.int32),
            [pltpu.VMEM((_C, _DIM), jnp.float32) for _ in range(_NBUF)],
            [pltpu.VMEM((_C, _DIM), jnp.float32) for _ in range(_NBUF)],
            [pltpu.SemaphoreType.DMA for _ in range(_NBUF)],
            [pltpu.SemaphoreType.DMA for _ in range(_NBUF)],
        ],
    )


_emb_kernel = _make_kernel()


def kernel(x, lut):
    idx = x.astype(jnp.int32).T.reshape(_NW, _CHUNKS, _C)
    out_t = _emb_kernel(idx, lut)
    return out_t.transpose(1, 0, 2)
